# speedup vs baseline: 18.8868x; 18.8868x over previous
import functools

import jax
import jax.numpy as jnp
from jax import lax
from jax.experimental import pallas as pl
from jax.experimental.pallas import tpu as pltpu
from jax.experimental.pallas import tpu_sc as plsc

V = 100000
L = 16
CV = 100
CE = 16
CP = 64
WP = 64
W = 3
T = L - W + 1
B, S = 4096, 50
N_TOK = B * S

RB = 1024
VPAD = ((V + 2 + RB - 1) // RB) * RB
N_BLOCKS = VPAD // RB


def _table_body(cc_ref, e_ref, bw_ref, bias_ref, wemb_ref, out_ref):
    cc = cc_ref[...]
    iota = lax.broadcasted_iota(jnp.int32, (RB, 128), 1)
    ce_parts = []
    for l in range(L):
        oh = (cc[:, l][:, None] == iota).astype(jnp.bfloat16)
        ce_parts.append(
            jnp.dot(oh, e_ref[...], preferred_element_type=jnp.float32))
    ceflat = jnp.concatenate(ce_parts, axis=1)
    conv = jnp.dot(ceflat.astype(jnp.bfloat16), bw_ref[...],
                   preferred_element_type=jnp.float32)
    conv = jnp.maximum(conv + bias_ref[0:1, :], 0.0)
    m = conv[:, 0:128]
    for j in range(1, T // 2):
        m = jnp.maximum(m, conv[:, j * 128:(j + 1) * 128])
    char_part = jnp.maximum(m[:, 0:64], m[:, 64:128])
    out_ref[...] = jnp.concatenate([char_part, wemb_ref[...]], axis=1)


def _build_table(cc_pad, e128, bigw, bias896, wemb_pad):
    return pl.pallas_call(
        _table_body,
        grid=(N_BLOCKS,),
        in_specs=[
            pl.BlockSpec((RB, L), lambda i: (i, 0)),
            pl.BlockSpec((128, CE), lambda i: (0, 0)),
            pl.BlockSpec((L * CE, T * CP), lambda i: (0, 0)),
            pl.BlockSpec((8, T * CP), lambda i: (0, 0)),
            pl.BlockSpec((RB, WP), lambda i: (i, 0)),
        ],
        out_specs=pl.BlockSpec((RB, CP + WP), lambda i: (i, 0)),
        out_shape=jax.ShapeDtypeStruct((VPAD, CP + WP), jnp.float32),
        compiler_params=pltpu.CompilerParams(
            dimension_semantics=("arbitrary",)),
    )(cc_pad, e128, bigw, bias896, wemb_pad)


_info = plsc.get_sparse_core_info()
_NC, _NS = _info.num_cores, _info.num_subcores
NW = _NC * _NS
B_PER_W = N_TOK // NW
CHUNK = 640
N_CHUNKS = B_PER_W // CHUNK


@functools.partial(
    pl.kernel,
    mesh=plsc.VectorSubcoreMesh(core_axis_name="c", subcore_axis_name="s"),
    out_type=jax.ShapeDtypeStruct((N_TOK, CP + WP), jnp.float32),
    scratch_types=[
        pltpu.VMEM((CHUNK,), jnp.int32),
        pltpu.VMEM((CHUNK, CP + WP), jnp.float32),
        pltpu.SemaphoreType.DMA,
    ],
)
def _gather_kernel(tab_hbm, idx_hbm, out_hbm, idx_v, rows_v, sem):
    wid = lax.axis_index("s") * _NC + lax.axis_index("c")
    base = wid * B_PER_W
    for j in range(N_CHUNKS):
        off = base + j * CHUNK
        pltpu.sync_copy(idx_hbm.at[pl.ds(off, CHUNK)], idx_v)
        pltpu.async_copy(tab_hbm.at[idx_v], rows_v, sem).wait()
        pltpu.sync_copy(rows_v, out_hbm.at[pl.ds(off, CHUNK)])


def kernel(x, char_codes, char_emb, conv_w, conv_b, word_emb):
    e128 = jnp.zeros((128, CE), jnp.float32).at[:CV].set(char_emb)
    e128 = e128.astype(jnp.bfloat16)
    shift = jnp.stack([jnp.eye(L, T, -k, dtype=jnp.float32) for k in range(W)])
    bigw = jnp.einsum('klt,oik->lito', shift, conv_w)
    bigw = bigw.reshape(L * CE, T * CP).astype(jnp.bfloat16)
    bias896 = jnp.broadcast_to(jnp.tile(conv_b, T), (8, T * CP))
    cc_pad = jnp.pad(char_codes, ((0, VPAD - (V + 2)), (0, 0)))
    wemb_pad = jnp.pad(word_emb, ((0, VPAD - (V + 2)), (0, 0)))
    idx = jnp.maximum(x.reshape(N_TOK), 0).astype(jnp.int32)

    tab = _build_table(cc_pad, e128, bigw, bias896, wemb_pad)
    out = _gather_kernel(tab, idx)
    return out.reshape(B, S, CP + WP)


# speedup vs baseline: 19.3375x; 1.0239x over previous
import functools

import jax
import jax.numpy as jnp
from jax import lax
from jax.experimental import pallas as pl
from jax.experimental.pallas import tpu as pltpu
from jax.experimental.pallas import tpu_sc as plsc

V = 100000
L = 16
CV = 100
CE = 16
CP = 64
WP = 64
W = 3
T = L - W + 1
B, S = 4096, 50
N_TOK = B * S

RB = 1024
N_BLOCKS = (V + 2 + RB - 1) // RB


def _table_body(cc_ref, e_ref, bw_ref, bias_ref, wemb_ref, out_ref):
    cc = cc_ref[...]
    iota = lax.broadcasted_iota(jnp.int32, (RB, 128), 1)
    ce_parts = []
    for l in range(L):
        oh = (cc[:, l][:, None] == iota).astype(jnp.bfloat16)
        ce_parts.append(
            jnp.dot(oh, e_ref[...], preferred_element_type=jnp.float32))
    ceflat = jnp.concatenate(ce_parts, axis=1)
    conv = jnp.dot(ceflat.astype(jnp.bfloat16), bw_ref[...],
                   preferred_element_type=jnp.float32)
    conv = jnp.maximum(conv + bias_ref[0:1, :], 0.0)
    m = conv[:, 0:128]
    for j in range(1, T // 2):
        m = jnp.maximum(m, conv[:, j * 128:(j + 1) * 128])
    char_part = jnp.maximum(m[:, 0:64], m[:, 64:128])
    out_ref[...] = jnp.concatenate([char_part, wemb_ref[...]], axis=1)


def _build_table(cc, e128, bigw, bias896, wemb):
    return pl.pallas_call(
        _table_body,
        grid=(N_BLOCKS,),
        in_specs=[
            pl.BlockSpec((RB, L), lambda i: (i, 0)),
            pl.BlockSpec((128, CE), lambda i: (0, 0)),
            pl.BlockSpec((L * CE, T * CP), lambda i: (0, 0)),
            pl.BlockSpec((8, T * CP), lambda i: (0, 0)),
            pl.BlockSpec((RB, WP), lambda i: (i, 0)),
        ],
        out_specs=pl.BlockSpec((RB, CP + WP), lambda i: (i, 0)),
        out_shape=jax.ShapeDtypeStruct((V + 2, CP + WP), jnp.float32),
        compiler_params=pltpu.CompilerParams(
            dimension_semantics=("arbitrary",)),
    )(cc, e128, bigw, bias896, wemb)


_info = plsc.get_sparse_core_info()
_NC, _NS = _info.num_cores, _info.num_subcores
NW = _NC * _NS
B_PER_W = N_TOK // NW
CHUNK = 640
N_CHUNKS = B_PER_W // CHUNK


@functools.partial(
    pl.kernel,
    mesh=plsc.VectorSubcoreMesh(core_axis_name="c", subcore_axis_name="s"),
    out_type=jax.ShapeDtypeStruct((N_TOK, CP + WP), jnp.float32),
    scratch_types=[
        pltpu.VMEM((CHUNK,), jnp.int32),
        pltpu.VMEM((CHUNK, CP + WP), jnp.float32),
        pltpu.SemaphoreType.DMA,
    ],
)
def _gather_kernel(tab_hbm, idx_hbm, out_hbm, idx_v, rows_v, sem):
    wid = lax.axis_index("s") * _NC + lax.axis_index("c")
    base = wid * B_PER_W
    for j in range(N_CHUNKS):
        off = base + j * CHUNK
        pltpu.sync_copy(idx_hbm.at[pl.ds(off, CHUNK)], idx_v)
        pltpu.async_copy(tab_hbm.at[idx_v], rows_v, sem).wait()
        pltpu.sync_copy(rows_v, out_hbm.at[pl.ds(off, CHUNK)])


def kernel(x, char_codes, char_emb, conv_w, conv_b, word_emb):
    e128 = jnp.zeros((128, CE), jnp.float32).at[:CV].set(char_emb)
    e128 = e128.astype(jnp.bfloat16)
    shift = jnp.stack([jnp.eye(L, T, -k, dtype=jnp.float32) for k in range(W)])
    bigw = jnp.einsum('klt,oik->lito', shift, conv_w)
    bigw = bigw.reshape(L * CE, T * CP).astype(jnp.bfloat16)
    bias896 = jnp.broadcast_to(jnp.tile(conv_b, T), (8, T * CP))
    idx = x.reshape(N_TOK)

    tab = _build_table(char_codes, e128, bigw, bias896, word_emb)
    out = _gather_kernel(tab, idx)
    return out.reshape(B, S, CP + WP)


# speedup vs baseline: 49.7561x; 2.5730x over previous
import functools

import jax
import jax.numpy as jnp
from jax import lax
from jax.experimental import pallas as pl
from jax.experimental.pallas import tpu as pltpu
from jax.experimental.pallas import tpu_sc as plsc

V = 100000
L = 16
CV = 100
CE = 16
CP = 64
WP = 64
W = 3
T = L - W + 1
Lh = L // 2
B, S = 4096, 50
N_TOK = B * S

RB = 2048
N_BLOCKS = (V + 2 + RB - 1) // RB


def _table_body(cc_ref, iosub_ref, eT_ref, bwT_ref, biasT_ref,
                wemb_ref, out_ref):
    ccT = cc_ref[...].T
    ohT = (ccT[:, None, :] == iosub_ref[...][None, :, :]).astype(jnp.bfloat16)
    ohT = ohT.reshape(L * 128, RB)
    ceT_parts = [
        jnp.dot(eT_ref[...], ohT[h * Lh * 128:(h + 1) * Lh * 128, :],
                preferred_element_type=jnp.float32)
        for h in range(2)
    ]
    ceT = jnp.concatenate(ceT_parts, axis=0).astype(jnp.bfloat16)
    convT = jnp.dot(bwT_ref[...], ceT,
                    preferred_element_type=jnp.float32)
    m = convT[0:128, :]
    for j in range(1, T // 2):
        m = jnp.maximum(m, convT[j * 128:(j + 1) * 128, :])
    m = m + biasT_ref[:, 0:1]
    char_partT = jnp.maximum(jnp.maximum(m[0:64, :], m[64:128, :]), 0.0)
    out_ref[...] = jnp.concatenate([char_partT.T, wemb_ref[...]], axis=1)


def _build_table(cc, iosub, ebigT, bigwT, biasT, wemb):
    return pl.pallas_call(
        _table_body,
        grid=(N_BLOCKS,),
        in_specs=[
            pl.BlockSpec((RB, L), lambda i: (i, 0)),
            pl.BlockSpec((128, RB), lambda i: (0, 0)),
            pl.BlockSpec((Lh * CE, Lh * 128), lambda i: (0, 0)),
            pl.BlockSpec((T * CP, L * CE), lambda i: (0, 0)),
            pl.BlockSpec((2 * CP, 8), lambda i: (0, 0)),
            pl.BlockSpec((RB, WP), lambda i: (i, 0)),
        ],
        out_specs=pl.BlockSpec((RB, CP + WP), lambda i: (i, 0)),
        out_shape=jax.ShapeDtypeStruct((V + 2, CP + WP), jnp.float32),
        compiler_params=pltpu.CompilerParams(
            dimension_semantics=("arbitrary",)),
    )(cc, iosub, ebigT, bigwT, biasT, wemb)


_info = plsc.get_sparse_core_info()
_NC, _NS = _info.num_cores, _info.num_subcores
NW = _NC * _NS
B_PER_W = N_TOK // NW
CHUNK = 400
N_CHUNKS = B_PER_W // CHUNK


@functools.partial(
    pl.kernel,
    mesh=plsc.VectorSubcoreMesh(core_axis_name="c", subcore_axis_name="s"),
    out_type=jax.ShapeDtypeStruct((N_TOK, CP + WP), jnp.float32),
    scratch_types=[
        pltpu.VMEM((B_PER_W,), jnp.int32),
        pltpu.VMEM((CHUNK, CP + WP), jnp.float32),
        pltpu.VMEM((CHUNK, CP + WP), jnp.float32),
        pltpu.SemaphoreType.DMA,
        pltpu.SemaphoreType.DMA,
        pltpu.SemaphoreType.DMA,
        pltpu.SemaphoreType.DMA,
    ],
)
def _gather_kernel(tab_hbm, idx_hbm, out_hbm, idx_v, rows0, rows1,
                   gsem0, gsem1, wsem0, wsem1):
    wid = lax.axis_index("s") * _NC + lax.axis_index("c")
    base = wid * B_PER_W
    rows = (rows0, rows1)
    gsems = (gsem0, gsem1)
    wsems = (wsem0, wsem1)
    pltpu.sync_copy(idx_hbm.at[pl.ds(base, B_PER_W)], idx_v)
    gh = [None] * N_CHUNKS
    wh = [None] * N_CHUNKS
    gh[0] = pltpu.async_copy(
        tab_hbm.at[idx_v.at[pl.ds(0, CHUNK)]], rows[0], gsems[0])
    for j in range(N_CHUNKS):
        p, q = j % 2, (j + 1) % 2
        if j + 1 < N_CHUNKS:
            if j >= 1:
                wh[j - 1].wait()
            gh[j + 1] = pltpu.async_copy(
                tab_hbm.at[idx_v.at[pl.ds((j + 1) * CHUNK, CHUNK)]],
                rows[q], gsems[q])
        gh[j].wait()
        wh[j] = pltpu.async_copy(
            rows[p], out_hbm.at[pl.ds(base + j * CHUNK, CHUNK)], wsems[p])
    wh[N_CHUNKS - 2].wait()
    wh[N_CHUNKS - 1].wait()


def kernel(x, char_codes, char_emb, conv_w, conv_b, word_emb):
    iosub = jnp.broadcast_to(
        jnp.arange(128, dtype=jnp.int32)[:, None], (128, RB))
    e128 = jnp.zeros((128, CE), jnp.float32).at[:CV].set(char_emb)
    ebigT = jnp.kron(jnp.eye(Lh, dtype=jnp.float32),
                     e128).astype(jnp.bfloat16).T
    shift = jnp.stack([jnp.eye(L, T, -k, dtype=jnp.float32) for k in range(W)])
    bigw = jnp.einsum('klt,oik->lito', shift, conv_w)
    bigwT = bigw.reshape(L * CE, T * CP).astype(jnp.bfloat16).T
    biasT = jnp.broadcast_to(jnp.tile(conv_b, 2)[:, None], (2 * CP, 8))
    idx = x.T.reshape(N_TOK)

    tab = _build_table(char_codes, iosub, ebigT, bigwT, biasT, word_emb)
    out = _gather_kernel(tab, idx)
    return out.reshape(S, B, CP + WP).transpose(1, 0, 2)


# speedup vs baseline: 51.0810x; 1.0266x over previous
import functools

import jax
import jax.numpy as jnp
from jax import lax
from jax.experimental import pallas as pl
from jax.experimental.pallas import tpu as pltpu
from jax.experimental.pallas import tpu_sc as plsc

V = 100000
L = 16
CV = 100
CE = 16
CP = 64
WP = 64
W = 3
T = L - W + 1
Lh = L // 2
B, S = 4096, 50
N_TOK = B * S

RB = 4096
N_BLOCKS = (V + 2 + RB - 1) // RB


def _table_body(cc_ref, iosub_ref, eT_ref, bwT_ref, biasT_ref,
                wemb_ref, out_ref):
    ccT = cc_ref[...].T
    ohT = (ccT[:, None, :] == iosub_ref[...][None, :, :]).astype(jnp.bfloat16)
    ohT = ohT.reshape(L * 128, RB)
    ceT_parts = [
        jnp.dot(eT_ref[...], ohT[h * Lh * 128:(h + 1) * Lh * 128, :],
                preferred_element_type=jnp.float32)
        for h in range(2)
    ]
    ceT = jnp.concatenate(ceT_parts, axis=0).astype(jnp.bfloat16)
    convT = jnp.dot(bwT_ref[...], ceT,
                    preferred_element_type=jnp.float32)
    m = convT[0:128, :]
    for j in range(1, T // 2):
        m = jnp.maximum(m, convT[j * 128:(j + 1) * 128, :])
    m = m + biasT_ref[:, 0:1]
    char_partT = jnp.maximum(jnp.maximum(m[0:64, :], m[64:128, :]), 0.0)
    out_ref[...] = jnp.concatenate([char_partT.T, wemb_ref[...]], axis=1)


def _build_table(cc, iosub, ebigT, bigwT, biasT, wemb):
    return pl.pallas_call(
        _table_body,
        grid=(N_BLOCKS,),
        in_specs=[
            pl.BlockSpec((RB, L), lambda i: (i, 0)),
            pl.BlockSpec((128, RB), lambda i: (0, 0)),
            pl.BlockSpec((Lh * CE, Lh * 128), lambda i: (0, 0)),
            pl.BlockSpec((T * CP, L * CE), lambda i: (0, 0)),
            pl.BlockSpec((2 * CP, 8), lambda i: (0, 0)),
            pl.BlockSpec((RB, WP), lambda i: (i, 0)),
        ],
        out_specs=pl.BlockSpec((RB, CP + WP), lambda i: (i, 0)),
        out_shape=jax.ShapeDtypeStruct((V + 2, CP + WP), jnp.float32),
        compiler_params=pltpu.CompilerParams(
            dimension_semantics=("arbitrary",)),
    )(cc, iosub, ebigT, bigwT, biasT, wemb)


_info = plsc.get_sparse_core_info()
_NC, _NS = _info.num_cores, _info.num_subcores
NW = _NC * _NS
B_PER_W = N_TOK // NW
CHUNK = 400
N_CHUNKS = B_PER_W // CHUNK


@functools.partial(
    pl.kernel,
    mesh=plsc.VectorSubcoreMesh(core_axis_name="c", subcore_axis_name="s"),
    out_type=jax.ShapeDtypeStruct((N_TOK, CP + WP), jnp.float32),
    scratch_types=[
        pltpu.VMEM((B_PER_W,), jnp.int32),
        pltpu.VMEM((CHUNK, CP + WP), jnp.float32),
        pltpu.VMEM((CHUNK, CP + WP), jnp.float32),
        pltpu.SemaphoreType.DMA,
        pltpu.SemaphoreType.DMA,
        pltpu.SemaphoreType.DMA,
        pltpu.SemaphoreType.DMA,
    ],
)
def _gather_kernel(tab_hbm, idx_hbm, out_hbm, idx_v, rows0, rows1,
                   gsem0, gsem1, wsem0, wsem1):
    wid = lax.axis_index("s") * _NC + lax.axis_index("c")
    base = wid * B_PER_W
    rows = (rows0, rows1)
    gsems = (gsem0, gsem1)
    wsems = (wsem0, wsem1)
    pltpu.sync_copy(idx_hbm.at[pl.ds(base, B_PER_W)], idx_v)
    gh = [None] * N_CHUNKS
    wh = [None] * N_CHUNKS
    gh[0] = pltpu.async_copy(
        tab_hbm.at[idx_v.at[pl.ds(0, CHUNK)]], rows[0], gsems[0])
    for j in range(N_CHUNKS):
        p, q = j % 2, (j + 1) % 2
        if j + 1 < N_CHUNKS:
            if j >= 1:
                wh[j - 1].wait()
            gh[j + 1] = pltpu.async_copy(
                tab_hbm.at[idx_v.at[pl.ds((j + 1) * CHUNK, CHUNK)]],
                rows[q], gsems[q])
        gh[j].wait()
        wh[j] = pltpu.async_copy(
            rows[p], out_hbm.at[pl.ds(base + j * CHUNK, CHUNK)], wsems[p])
    wh[N_CHUNKS - 2].wait()
    wh[N_CHUNKS - 1].wait()


def kernel(x, char_codes, char_emb, conv_w, conv_b, word_emb):
    iosub = jnp.broadcast_to(
        jnp.arange(128, dtype=jnp.int32)[:, None], (128, RB))
    e128 = jnp.zeros((128, CE), jnp.float32).at[:CV].set(char_emb)
    ebigT = jnp.kron(jnp.eye(Lh, dtype=jnp.float32),
                     e128).astype(jnp.bfloat16).T
    shift = jnp.stack([jnp.eye(L, T, -k, dtype=jnp.float32) for k in range(W)])
    bigw = jnp.einsum('klt,oik->lito', shift, conv_w)
    bigwT = bigw.reshape(L * CE, T * CP).astype(jnp.bfloat16).T
    biasT = jnp.broadcast_to(jnp.tile(conv_b, 2)[:, None], (2 * CP, 8))
    idx = x.T.reshape(N_TOK)

    tab = _build_table(char_codes, iosub, ebigT, bigwT, biasT, word_emb)
    out = _gather_kernel(tab, idx)
    return out.reshape(S, B, CP + WP).transpose(1, 0, 2)


# speedup vs baseline: 51.0841x; 1.0001x over previous
import functools

import jax
import jax.numpy as jnp
from jax import lax
from jax.experimental import pallas as pl
from jax.experimental.pallas import tpu as pltpu
from jax.experimental.pallas import tpu_sc as plsc

V = 100000
L = 16
CV = 100
CE = 16
CP = 64
WP = 64
W = 3
T = L - W + 1
Lh = L // 2
B, S = 4096, 50
N_TOK = B * S

RB = 4096
N_BLOCKS = (V + 2 + RB - 1) // RB


def _table_body(cc_ref, iosub_ref, eT_ref, bwT_ref, biasT_ref,
                wemb_ref, out_ref):
    ccT = cc_ref[...].T
    ohT = (ccT[:, None, :] == iosub_ref[...][None, :, :]).astype(jnp.bfloat16)
    ohT = ohT.reshape(L * 128, RB)
    ceT_parts = [
        jnp.dot(eT_ref[...], ohT[h * Lh * 128:(h + 1) * Lh * 128, :],
                preferred_element_type=jnp.float32)
        for h in range(2)
    ]
    ceT = jnp.concatenate(ceT_parts, axis=0).astype(jnp.bfloat16)
    convT = jnp.dot(bwT_ref[...], ceT,
                    preferred_element_type=jnp.float32)
    m = convT[0:128, :]
    for j in range(1, T // 2):
        m = jnp.maximum(m, convT[j * 128:(j + 1) * 128, :])
    m = m + biasT_ref[:, 0:1]
    char_partT = jnp.maximum(jnp.maximum(m[0:64, :], m[64:128, :]), 0.0)
    out_ref[...] = jnp.concatenate([char_partT.T, wemb_ref[...]], axis=1)


def _build_table(cc, iosub, ebigT, bigwT, biasT, wemb):
    return pl.pallas_call(
        _table_body,
        grid=(N_BLOCKS,),
        in_specs=[
            pl.BlockSpec((RB, L), lambda i: (i, 0)),
            pl.BlockSpec((128, RB), lambda i: (0, 0)),
            pl.BlockSpec((Lh * CE, Lh * 128), lambda i: (0, 0)),
            pl.BlockSpec((T * CP, L * CE), lambda i: (0, 0)),
            pl.BlockSpec((2 * CP, 8), lambda i: (0, 0)),
            pl.BlockSpec((RB, WP), lambda i: (i, 0)),
        ],
        out_specs=pl.BlockSpec((RB, CP + WP), lambda i: (i, 0)),
        out_shape=jax.ShapeDtypeStruct((V + 2, CP + WP), jnp.float32),
        compiler_params=pltpu.CompilerParams(
            dimension_semantics=("arbitrary",)),
    )(cc, iosub, ebigT, bigwT, biasT, wemb)


_info = plsc.get_sparse_core_info()
_NC, _NS = _info.num_cores, _info.num_subcores
NW = _NC * _NS
B_PER_W = N_TOK // NW
CHUNK = 320
N_CHUNKS = B_PER_W // CHUNK
NBUF = 3


@functools.partial(
    pl.kernel,
    mesh=plsc.VectorSubcoreMesh(core_axis_name="c", subcore_axis_name="s"),
    out_type=jax.ShapeDtypeStruct((N_TOK, CP + WP), jnp.float32),
    scratch_types=(
        [pltpu.VMEM((B_PER_W,), jnp.int32)]
        + [pltpu.VMEM((CHUNK, CP + WP), jnp.float32)] * NBUF
        + [pltpu.SemaphoreType.DMA] * (2 * NBUF)
    ),
)
def _gather_kernel(tab_hbm, idx_hbm, out_hbm, idx_v, *bufs):
    rows = bufs[:NBUF]
    gsems = bufs[NBUF:2 * NBUF]
    wsems = bufs[2 * NBUF:]
    wid = lax.axis_index("s") * _NC + lax.axis_index("c")
    base = wid * B_PER_W
    pltpu.sync_copy(idx_hbm.at[pl.ds(base, B_PER_W)], idx_v)
    gh = [None] * N_CHUNKS
    wh = [None] * N_CHUNKS

    def start_gather(j):
        return pltpu.async_copy(
            tab_hbm.at[idx_v.at[pl.ds(j * CHUNK, CHUNK)]],
            rows[j % NBUF], gsems[j % NBUF])

    for j in range(NBUF - 1):
        gh[j] = start_gather(j)
    for j in range(N_CHUNKS):
        p = j % NBUF
        n = j + NBUF - 1
        if n < N_CHUNKS:
            if n - NBUF >= 0:
                wh[n - NBUF].wait()
            gh[n] = start_gather(n)
        gh[j].wait()
        wh[j] = pltpu.async_copy(
            rows[p], out_hbm.at[pl.ds(base + j * CHUNK, CHUNK)], wsems[p])
    for j in range(N_CHUNKS - NBUF, N_CHUNKS):
        wh[j].wait()


def kernel(x, char_codes, char_emb, conv_w, conv_b, word_emb):
    iosub = jnp.broadcast_to(
        jnp.arange(128, dtype=jnp.int32)[:, None], (128, RB))
    e128 = jnp.zeros((128, CE), jnp.float32).at[:CV].set(char_emb)
    ebigT = jnp.kron(jnp.eye(Lh, dtype=jnp.float32),
                     e128).astype(jnp.bfloat16).T
    shift = jnp.stack([jnp.eye(L, T, -k, dtype=jnp.float32) for k in range(W)])
    bigw = jnp.einsum('klt,oik->lito', shift, conv_w)
    bigwT = bigw.reshape(L * CE, T * CP).astype(jnp.bfloat16).T
    biasT = jnp.broadcast_to(jnp.tile(conv_b, 2)[:, None], (2 * CP, 8))
    idx = x.T.reshape(N_TOK)

    tab = _build_table(char_codes, iosub, ebigT, bigwT, biasT, word_emb)
    out = _gather_kernel(tab, idx)
    return out.reshape(S, B, CP + WP).transpose(1, 0, 2)
